# Initial kernel scaffold; baseline (speedup 1.0000x reference)
import jax
import jax.numpy as jnp
from jax.experimental import pallas as pl


def kernel(data_a, data_b, source_table, target_table, fc_W, fc_b):
    raise NotImplementedError("write your pallas kernel here")


# speedup vs baseline: 2.7658x; 2.7658x over previous
import functools

import jax
import jax.numpy as jnp
from jax import lax
from jax.experimental import pallas as pl
from jax.experimental.pallas import tpu as pltpu
from jax.experimental.pallas import tpu_sc as plsc

B = 16384
DIM_E = 128
NUM_CORES = 2
NUM_SUBCORES = 16
NW = NUM_CORES * NUM_SUBCORES
BPW = B // NW
IDX_CHUNK = 128
NCHUNK = BPW // IDX_CHUNK


def _sc_gather(table, idx2):
    mesh = plsc.VectorSubcoreMesh(core_axis_name="c", subcore_axis_name="s")

    @functools.partial(
        pl.kernel,
        out_type=jax.ShapeDtypeStruct((B, DIM_E), jnp.float32),
        mesh=mesh,
        scratch_types=[
            pltpu.VMEM((NCHUNK, IDX_CHUNK), jnp.int32),
            pltpu.VMEM((BPW, DIM_E), jnp.float32),
            pltpu.SemaphoreType.DMA,
        ],
    )
    def gather_rows(table_hbm, idx_hbm, out_hbm, idx_v, rows_v, sem):
        wid = lax.axis_index("s") * NUM_CORES + lax.axis_index("c")
        base = wid * BPW
        pltpu.sync_copy(idx_hbm.at[pl.ds(wid * NCHUNK, NCHUNK)], idx_v)
        copies = []
        for j in range(NCHUNK):
            copies.append(
                pltpu.make_async_copy(
                    table_hbm.at[idx_v.at[j]],
                    rows_v.at[pl.ds(j * IDX_CHUNK, IDX_CHUNK)],
                    sem,
                )
            )
            copies[-1].start()
        for c in copies:
            c.wait()
        pltpu.sync_copy(rows_v, out_hbm.at[pl.ds(base, BPW)])

    return gather_rows(table, idx2)


def _tc_body(rows_ref, w_ref, b_ref, preds_ref, ents_ref):
    rows = rows_ref[...]
    ss = jnp.sum(rows * rows, axis=1, keepdims=True)
    inv = lax.rsqrt(jnp.maximum(ss, 1e-24))
    ents = rows * inv
    ents_ref[...] = ents
    preds_ref[...] = (
        lax.dot_general(
            ents, w_ref[...], (((1,), (1,)), ((), ())),
            preferred_element_type=jnp.float32,
        )
        + b_ref[...]
    )


def _tc_normalize_project(rows, fc_W, fc_b2):
    blk = 2048
    grid = (B // blk,)
    return pl.pallas_call(
        _tc_body,
        grid=grid,
        in_specs=[
            pl.BlockSpec((blk, DIM_E), lambda i: (i, 0)),
            pl.BlockSpec((DIM_E, DIM_E), lambda i: (0, 0)),
            pl.BlockSpec((1, DIM_E), lambda i: (0, 0)),
        ],
        out_specs=[
            pl.BlockSpec((blk, DIM_E), lambda i: (i, 0)),
            pl.BlockSpec((blk, DIM_E), lambda i: (i, 0)),
        ],
        out_shape=[
            jax.ShapeDtypeStruct((B, DIM_E), jnp.float32),
            jax.ShapeDtypeStruct((B, DIM_E), jnp.float32),
        ],
        compiler_params=pltpu.CompilerParams(
            dimension_semantics=("parallel",),
        ),
    )(rows, fc_W, fc_b2)


def kernel(data_a, data_b, source_table, target_table, fc_W, fc_b):
    idx2 = data_b.astype(jnp.int32).reshape(NW * NCHUNK, IDX_CHUNK)
    rows = _sc_gather(target_table, idx2)
    target_preds, target_ents = _tc_normalize_project(
        rows, fc_W, fc_b.reshape(1, DIM_E)
    )
    return (target_preds, target_ents)
